# Initial kernel scaffold; baseline (speedup 1.0000x reference)
import jax
import jax.numpy as jnp
from jax.experimental import pallas as pl


def kernel(hidden_states, route_prior, W_proj, b_proj, block_centers, down_w, up_w, route_prior_scale, residual_scale):
    raise NotImplementedError("write your pallas kernel here")


# speedup vs baseline: 3.5500x; 3.5500x over previous
import jax
import jax.numpy as jnp
from jax.experimental import pallas as pl

HS = 2048
NB = 32
RANK = 8
GRID_N = 8
SIGMA = 1.0
ROW_TILE = 512
QPAD = 128


def _fused_kernel(x_ref, prior_ref, wproj_ref, bias_ref, centers_ref,
                  down_ref, up_ref, scal_ref, out_ref):
    x = x_ref[...]
    rps = scal_ref[0, 0]
    res = scal_ref[0, 1]

    mu = jnp.mean(x, axis=1, keepdims=True)
    var = jnp.mean(x * x, axis=1, keepdims=True) - mu * mu
    ln = (x - mu) * jax.lax.rsqrt(var + 1e-5)

    qraw = jnp.dot(ln, wproj_ref[...], preferred_element_type=jnp.float32)
    qraw = qraw + bias_ref[...]
    q = jax.nn.sigmoid(qraw) * float(GRID_N - 1)
    col = jax.lax.broadcasted_iota(jnp.int32, q.shape, 1)
    q = jnp.where(col < 3, q, 0.0)
    c = centers_ref[...]
    qn = jnp.sum(q * q, axis=1, keepdims=True)
    cn = jnp.sum(c * c, axis=1)[None, :]
    qc = jnp.dot(q, c.T, preferred_element_type=jnp.float32)
    d2 = qn + cn - 2.0 * qc
    spatial = jnp.exp(-d2 / (2.0 * SIGMA * SIGMA))

    prior = jnp.maximum(prior_ref[...], 0.0)
    prior = prior / jnp.maximum(jnp.sum(prior, axis=1, keepdims=True), 1e-6)
    prior_bias = jnp.clip(jnp.log(prior + 1e-6), -6.0, 0.0)
    fused = spatial + rps * prior_bias

    iota = jax.lax.broadcasted_iota(jnp.int32, fused.shape, 1)
    m1 = jnp.max(fused, axis=1, keepdims=True)
    i1 = jnp.min(jnp.where(fused == m1, iota, NB), axis=1, keepdims=True)
    oh1 = iota == i1
    masked = jnp.where(oh1, -jnp.inf, fused)
    m2 = jnp.max(masked, axis=1, keepdims=True)
    i2 = jnp.min(jnp.where(masked == m2, iota, NB), axis=1, keepdims=True)
    oh2 = iota == i2
    e2 = jnp.exp(m2 - m1)
    w1 = 1.0 / (1.0 + e2)
    w2 = e2 * w1
    wfull = jnp.where(oh1, w1, 0.0) + jnp.where(oh2, w2, 0.0)

    z = jnp.dot(x, down_ref[...], preferred_element_type=jnp.float32)
    wtile = jnp.concatenate([wfull] * RANK, axis=1)
    delta = jnp.dot(z * wtile, up_ref[...], preferred_element_type=jnp.float32)
    out_ref[...] = x + res * delta


def kernel(hidden_states, route_prior, W_proj, b_proj, block_centers, down_w,
           up_w, route_prior_scale, residual_scale):
    b, s, h = hidden_states.shape
    rows = b * s
    flat = hidden_states.reshape(rows, h)

    wproj_pad = jnp.zeros((h, QPAD), jnp.float32).at[:, :3].set(W_proj.T)
    bias_pad = jnp.zeros((1, QPAD), jnp.float32).at[0, :3].set(b_proj)
    centers_pad = jnp.zeros((NB, QPAD), jnp.float32).at[:, :3].set(block_centers)
    down_all = down_w.transpose(1, 2, 0).reshape(h, RANK * NB)
    up_all = up_w.transpose(1, 0, 2).reshape(RANK * NB, h)
    scal = jnp.stack([route_prior_scale, residual_scale]).reshape(1, 2).astype(jnp.float32)

    grid = rows // ROW_TILE

    out = pl.pallas_call(
        _fused_kernel,
        grid=(grid,),
        in_specs=[
            pl.BlockSpec((ROW_TILE, h), lambda i: (i, 0)),
            pl.BlockSpec((ROW_TILE, NB), lambda i: (i, 0)),
            pl.BlockSpec((h, QPAD), lambda i: (0, 0)),
            pl.BlockSpec((1, QPAD), lambda i: (0, 0)),
            pl.BlockSpec((NB, QPAD), lambda i: (0, 0)),
            pl.BlockSpec((h, RANK * NB), lambda i: (0, 0)),
            pl.BlockSpec((RANK * NB, h), lambda i: (0, 0)),
            pl.BlockSpec((1, 2), lambda i: (0, 0)),
        ],
        out_specs=pl.BlockSpec((ROW_TILE, h), lambda i: (i, 0)),
        out_shape=jax.ShapeDtypeStruct((rows, h), jnp.float32),
    )(flat, route_prior, wproj_pad, bias_pad, centers_pad, down_all, up_all, scal)

    return out.reshape(b, s, h)
